# baseline (device time: 21066 ns/iter reference)
import jax
import jax.numpy as jnp
from jax import lax
from jax.experimental import pallas as pl
from jax.experimental.pallas import tpu as pltpu

P = 16
PLANES = 4
PER = 4


def kernel(x, w_mat):
    m_per, k = x.shape
    _, n = w_mat.shape
    n_per = n // P
    m = m_per * P
    n_pl = n // PLANES
    m_pl = m // PLANES

    def body(x_ref, w_ref, out_ref, y_bf, recv_buf,
             send_sems, recv_sems, ready_sems):
        my = lax.axis_index("i")
        my_pl = my // PER
        my_in_pl = my % PER

        for d in range(1, P):
            pl.semaphore_signal(
                ready_sems.at[my], inc=1,
                device_id=((my + d) % P,),
                device_id_type=pl.DeviceIdType.MESH,
            )

        barrier_sem = pltpu.get_barrier_semaphore()
        for nbr in [(my + 1) % P, (my - 1) % P]:
            pl.semaphore_signal(
                barrier_sem, inc=1,
                device_id=(nbr,), device_id_type=pl.DeviceIdType.MESH,
            )
        pl.semaphore_wait(barrier_sem, 2)

        for off in (2, 3, 1, 0):
            q = (my_pl + off) % PLANES
            y_pl = jnp.dot(
                x_ref[...],
                w_ref[:, pl.ds(q * n_pl, n_pl)],
                preferred_element_type=jnp.float32,
            ).astype(jnp.bfloat16)
            y_bf[:, pl.ds(q * n_pl, n_pl)] = y_pl
            for j in range(PER):
                dst = q * PER + (my_in_pl + j) % PER
                if off == 0 and j == 0:
                    recv_buf[pl.ds(my * m_per, m_per), :] = y_bf[
                        :, pl.ds(my * n_per, n_per)
                    ]
                    continue
                pl.semaphore_wait(ready_sems.at[dst], 1)
                rdma = pltpu.make_async_remote_copy(
                    src_ref=y_bf.at[:, pl.ds(dst * n_per, n_per)],
                    dst_ref=recv_buf.at[pl.ds(my * m_per, m_per), :],
                    send_sem=send_sems.at[dst],
                    recv_sem=recv_sems.at[my],
                    device_id=(dst,),
                    device_id_type=pl.DeviceIdType.MESH,
                )
                rdma.start()

        for off in (0, 1, 3, 2):
            q = (my_pl + off) % PLANES
            for j in range(PER):
                if off == 0 and j == 0:
                    continue
                src = q * PER + (my_in_pl + j) % PER
                recv = pltpu.make_async_remote_copy(
                    src_ref=y_bf.at[:, pl.ds(0, n_per)],
                    dst_ref=recv_buf.at[pl.ds(src * m_per, m_per), :],
                    send_sem=send_sems.at[0],
                    recv_sem=recv_sems.at[src],
                    device_id=(my,),
                    device_id_type=pl.DeviceIdType.MESH,
                )
                recv.wait_recv()
            out_ref[pl.ds(q * m_pl, m_pl), :] = recv_buf[
                pl.ds(q * m_pl, m_pl), :
            ].astype(jnp.float32)

        for d in range(1, P):
            dst = (my + d) % P
            send = pltpu.make_async_remote_copy(
                src_ref=y_bf.at[:, pl.ds(0, n_per)],
                dst_ref=recv_buf.at[pl.ds(0, m_per), :],
                send_sem=send_sems.at[dst],
                recv_sem=recv_sems.at[my],
                device_id=(my,),
                device_id_type=pl.DeviceIdType.MESH,
            )
            send.wait_send()

    return pl.pallas_call(
        body,
        out_shape=jax.ShapeDtypeStruct((m, n_per), jnp.float32),
        in_specs=[
            pl.BlockSpec(memory_space=pltpu.VMEM),
            pl.BlockSpec(memory_space=pltpu.VMEM),
        ],
        out_specs=pl.BlockSpec(memory_space=pltpu.VMEM),
        scratch_shapes=[
            pltpu.VMEM((m_per, n), jnp.bfloat16),
            pltpu.VMEM((m, n_per), jnp.bfloat16),
            pltpu.SemaphoreType.DMA((P,)),
            pltpu.SemaphoreType.DMA((P,)),
            pltpu.SemaphoreType.REGULAR((P,)),
        ],
        compiler_params=pltpu.CompilerParams(collective_id=0),
    )(x, w_mat)
